# baseline (speedup 1.0000x reference)
import functools

import jax
import jax.numpy as jnp
from jax import lax
from jax.experimental import pallas as pl
from jax.experimental.pallas import tpu as pltpu
from jax.experimental.pallas import tpu_sc as plsc

N = 10000
NP = 10240
D = 128
E = 320000
NC = 2
NS = 16
NW = NC * NS
EPW = E // NW
EPW2 = 10240
CH = 128
NCHUNK = EPW2 // CH
G = 10
NGRP = NCHUNK // G
RPT = NP // NS
DEGW = 16
L = 16
HR = NP // D

_mesh = plsc.VectorSubcoreMesh(core_axis_name="c", subcore_axis_name="s")


@functools.partial(
    pl.kernel,
    mesh=_mesh,
    out_type=jax.ShapeDtypeStruct((NC, NP // 8, D), jnp.float32),
    scratch_types=[
        pltpu.VMEM((EPW2 // D, D), jnp.int32),
        pltpu.VMEM((HR, D), jnp.float32),
        pltpu.VMEM((HR,), jnp.int32),
        pltpu.VMEM((HR, D), jnp.float32),
        pltpu.VMEM((HR, D), jnp.float32),
        pltpu.VMEM_SHARED((HR, D), jnp.float32),
        pltpu.VMEM_SHARED((NP // 8, D), jnp.float32),
    ],
    compiler_params=pltpu.CompilerParams(needs_layout_passes=False),
)
def _deg_call(dst2d_hbm, zeros_hbm, out_hbm, idx_v, hist_v, iota_v, red_v,
              bcast_v, acc_sh, stage_sh):
    cid = lax.axis_index("c")
    sid = lax.axis_index("s")
    wid = cid * NS + sid
    pltpu.sync_copy(dst2d_hbm.at[wid], idx_v)

    @pl.when(sid == 0)
    def _():
        pltpu.sync_copy(zeros_hbm.at[pl.ds(0, HR)], acc_sh)

    for k in range(HR // L):
        iota_v[pl.ds(k * L, L)] = lax.iota(jnp.int32, L) + (k * L)

    def z16(i, carry):
        for k in range(D // L):
            hist_v[i, pl.ds(k * L, L)] = jnp.zeros((L,), jnp.float32)
        return carry

    lax.fori_loop(0, HR, z16, 0)

    ones16 = jnp.full((L,), 1.0, jnp.float32)

    def body(i, carry):
        for k in range(D // L):
            idx16 = idx_v[i, pl.ds(k * L, L)]
            hi = lax.shift_right_logical(idx16, 7)
            lo = lax.bitwise_and(idx16, 127)
            plsc.addupdate_scatter(hist_v, [hi, lo], ones16)
        return carry

    lax.fori_loop(0, EPW2 // D, body, 0)
    plsc.subcore_barrier()
    pltpu.sync_copy(hist_v, acc_sh.at[iota_v], add=True)
    plsc.subcore_barrier()
    pltpu.sync_copy(acc_sh, red_v)

    def bbody(i, carry):
        n0 = sid * RPT + i * L
        v = red_v[n0 // D, pl.ds(n0 % D, L)]
        for l in range(L):
            f = (i * L + l) * DEGW
            bcast_v[f // D, pl.ds(f % D, L)] = jnp.full((L,), v[l],
                                                        jnp.float32)
        return carry

    lax.fori_loop(0, RPT // L, bbody, 0)
    pltpu.sync_copy(bcast_v, stage_sh.at[pl.ds(sid * HR, HR)])
    plsc.subcore_barrier()

    @pl.when(sid == 0)
    def _():
        pltpu.sync_copy(stage_sh, out_hbm.at[cid])


CHS = 80
NCHS = EPW // CHS


NBODY = NCHS - 1


@functools.partial(
    pl.kernel,
    mesh=_mesh,
    out_type=jax.ShapeDtypeStruct((NC, NP, D), jnp.float32),
    scratch_types=[
        pltpu.VMEM((4, CHS), jnp.int32),
        pltpu.VMEM((4, CHS), jnp.int32),
        pltpu.VMEM((4, CHS, D), jnp.float32),
        pltpu.VMEM_SHARED((NP, D), jnp.float32),
    ]
    + [pltpu.SemaphoreType.DMA] * 8,
)
def _scatter_call(tab_hbm, src1d_hbm, dst1d_hbm, zeros_hbm, out_hbm,
                  si_v, di_v, rows_v, acc_sh, *sems):
    semg = sems[:4]
    sems_ = sems[4:]
    cid = lax.axis_index("c")
    sid = lax.axis_index("s")
    wid = cid * NS + sid
    pltpu.sync_copy(zeros_hbm, acc_sh.at[pl.ds(sid * RPT, RPT)])
    base0 = wid * EPW
    for b in range(2):
        pltpu.sync_copy(src1d_hbm.at[pl.ds(base0 + b * CHS, CHS)],
                        si_v.at[b])
        pltpu.sync_copy(dst1d_hbm.at[pl.ds(base0 + b * CHS, CHS)],
                        di_v.at[b])
    plsc.subcore_barrier()

    pltpu.async_copy(tab_hbm.at[si_v.at[0]], rows_v.at[0], semg[0])
    pltpu.async_copy(tab_hbm.at[si_v.at[1]], rows_v.at[1], semg[1])

    def body(k, carry):
        for j in range(4):
            c = k * 4 + j
            s = (j + 2) % 4

            def drain():
                pltpu.make_async_copy(rows_v.at[s],
                                      acc_sh.at[di_v.at[0]],
                                      sems_[s]).wait()

            if j < 2:
                @pl.when(k > 0)
                def _():
                    drain()
            else:
                drain()

            @pl.when(c + 2 < NCHS)
            def _():
                pltpu.sync_copy(
                    src1d_hbm.at[pl.ds(base0 + (c + 2) * CHS, CHS)],
                    si_v.at[s])
                pltpu.sync_copy(
                    dst1d_hbm.at[pl.ds(base0 + (c + 2) * CHS, CHS)],
                    di_v.at[s])

            pltpu.make_async_copy(tab_hbm.at[si_v.at[0]], rows_v.at[j],
                                  semg[j]).wait()
            pltpu.async_copy(rows_v.at[j], acc_sh.at[di_v.at[j]], sems_[j],
                             add=True)

            @pl.when(c + 2 < NCHS)
            def _():
                pltpu.async_copy(tab_hbm.at[si_v.at[s]], rows_v.at[s],
                                 semg[s])
        return carry

    lax.fori_loop(0, NBODY // 4, body, 0)
    pltpu.make_async_copy(tab_hbm.at[si_v.at[0]], rows_v.at[0],
                          semg[0]).wait()
    pltpu.sync_copy(rows_v.at[0], acc_sh.at[di_v.at[0]], add=True)
    pltpu.make_async_copy(rows_v.at[2], acc_sh.at[di_v.at[0]],
                          sems_[2]).wait()
    pltpu.make_async_copy(rows_v.at[3], acc_sh.at[di_v.at[0]],
                          sems_[3]).wait()
    plsc.subcore_barrier()
    pltpu.sync_copy(acc_sh.at[pl.ds(sid * RPT, RPT)],
                    out_hbm.at[cid, pl.ds(sid * RPT, RPT)])


BLK = 400


def _dis_from(degp_ref):
    deg = degp_ref[0, :, 0:1] + degp_ref[1, :, 0:1]
    return jnp.where(deg > 0, lax.rsqrt(deg), 0.0)


def _mm_scale_body(x_ref, w_ref, degp_ref, o_ref):
    dis = _dis_from(degp_ref)
    xw = jnp.dot(x_ref[...], w_ref[...],
                 preferred_element_type=jnp.float32,
                 precision=lax.Precision.HIGHEST)
    o_ref[...] = xw * dis


def _combine_mm_body(p_ref, degp_ref, b_ref, w_ref, o_ref):
    dis = _dis_from(degp_ref)
    h = jnp.maximum((p_ref[0] + p_ref[1]) * dis + b_ref[...], 0.0)
    o_ref[...] = jnp.dot(h, w_ref[...],
                         preferred_element_type=jnp.float32,
                         precision=lax.Precision.HIGHEST) * dis


def _final_body(p_ref, degp_ref, b_ref, o_ref):
    dis = _dis_from(degp_ref)
    o_ref[...] = (p_ref[0] + p_ref[1]) * dis + b_ref[...]


_mm_scale = pl.pallas_call(
    _mm_scale_body,
    grid=(N // BLK,),
    in_specs=[
        pl.BlockSpec((BLK, D), lambda i: (i, 0)),
        pl.BlockSpec((D, D), lambda i: (0, 0)),
        pl.BlockSpec((NC, BLK, DEGW), lambda i: (0, i, 0)),
    ],
    out_specs=pl.BlockSpec((BLK, D), lambda i: (i, 0)),
    out_shape=jax.ShapeDtypeStruct((N, D), jnp.float32),
)

_combine_mm = pl.pallas_call(
    _combine_mm_body,
    grid=(N // BLK,),
    in_specs=[
        pl.BlockSpec((NC, BLK, D), lambda i: (0, i, 0)),
        pl.BlockSpec((NC, BLK, DEGW), lambda i: (0, i, 0)),
        pl.BlockSpec((1, D), lambda i: (0, 0)),
        pl.BlockSpec((D, D), lambda i: (0, 0)),
    ],
    out_specs=pl.BlockSpec((BLK, D), lambda i: (i, 0)),
    out_shape=jax.ShapeDtypeStruct((N, D), jnp.float32),
)

_final = pl.pallas_call(
    _final_body,
    grid=(N // BLK,),
    in_specs=[
        pl.BlockSpec((NC, BLK, D), lambda i: (0, i, 0)),
        pl.BlockSpec((NC, BLK, DEGW), lambda i: (0, i, 0)),
        pl.BlockSpec((1, D), lambda i: (0, 0)),
    ],
    out_specs=pl.BlockSpec((BLK, D), lambda i: (i, 0)),
    out_shape=jax.ShapeDtypeStruct((N, D), jnp.float32),
)


def kernel(x, edge_index, W1, b1, W2, b2):
    src = edge_index[0].astype(jnp.int32).reshape(NW, EPW)
    dst = edge_index[1].astype(jnp.int32).reshape(NW, EPW)
    padn = EPW2 - EPW
    srcp = jnp.concatenate(
        [src, jnp.zeros((NW, padn), jnp.int32)], axis=1)
    pad_rows = N + jnp.arange(padn, dtype=jnp.int32)
    dstp = jnp.concatenate(
        [dst, jnp.broadcast_to(pad_rows, (NW, padn))], axis=1)
    src1d = edge_index[0].astype(jnp.int32)
    dst1d = edge_index[1].astype(jnp.int32)
    dst2d = dstp.reshape(NW, EPW2 // D, D)
    zeros_d = jnp.zeros((RPT, D), jnp.float32)
    b1r = b1.reshape(1, D)
    b2r = b2.reshape(1, D)

    degp = _deg_call(dst2d, zeros_d).reshape(NC, NP, DEGW)
    xws1 = _mm_scale(x, W1, degp)
    p1 = _scatter_call(xws1, src1d, dst1d, zeros_d)
    xws2 = _combine_mm(p1, degp, b1r, W2)
    p2 = _scatter_call(xws2, src1d, dst1d, zeros_d)
    return _final(p2, degp, b2r)


# speedup vs baseline: 1.0011x; 1.0011x over previous
import functools

import jax
import jax.numpy as jnp
from jax import lax
from jax.experimental import pallas as pl
from jax.experimental.pallas import tpu as pltpu
from jax.experimental.pallas import tpu_sc as plsc

N = 10000
NP = 10240
D = 128
E = 320000
NC = 2
NS = 16
NW = NC * NS
EPW = E // NW
EPW2 = 10240
CH = 128
NCHUNK = EPW2 // CH
G = 10
NGRP = NCHUNK // G
RPT = NP // NS
DEGW = 16
L = 16
HR = NP // D

_mesh = plsc.VectorSubcoreMesh(core_axis_name="c", subcore_axis_name="s")


@functools.partial(
    pl.kernel,
    mesh=_mesh,
    out_type=jax.ShapeDtypeStruct((NC, NP // 8, D), jnp.float32),
    scratch_types=[
        pltpu.VMEM((EPW2 // D, D), jnp.int32),
        pltpu.VMEM((HR, D), jnp.float32),
        pltpu.VMEM((HR,), jnp.int32),
        pltpu.VMEM((HR, D), jnp.float32),
        pltpu.VMEM((HR, D), jnp.float32),
        pltpu.VMEM_SHARED((HR, D), jnp.float32),
        pltpu.VMEM_SHARED((NP // 8, D), jnp.float32),
    ],
    compiler_params=pltpu.CompilerParams(needs_layout_passes=False),
)
def _deg_call(dst2d_hbm, zeros_hbm, out_hbm, idx_v, hist_v, iota_v, red_v,
              bcast_v, acc_sh, stage_sh):
    cid = lax.axis_index("c")
    sid = lax.axis_index("s")
    wid = cid * NS + sid
    pltpu.sync_copy(dst2d_hbm.at[wid], idx_v)

    @pl.when(sid == 0)
    def _():
        pltpu.sync_copy(zeros_hbm.at[pl.ds(0, HR)], acc_sh)

    for k in range(HR // L):
        iota_v[pl.ds(k * L, L)] = lax.iota(jnp.int32, L) + (k * L)

    def z16(i, carry):
        for k in range(D // L):
            hist_v[i, pl.ds(k * L, L)] = jnp.zeros((L,), jnp.float32)
        return carry

    lax.fori_loop(0, HR, z16, 0)

    ones16 = jnp.full((L,), 1.0, jnp.float32)

    def body(i, carry):
        for k in range(D // L):
            idx16 = idx_v[i, pl.ds(k * L, L)]
            hi = lax.shift_right_logical(idx16, 7)
            lo = lax.bitwise_and(idx16, 127)
            plsc.addupdate_scatter(hist_v, [hi, lo], ones16)
        return carry

    lax.fori_loop(0, EPW2 // D, body, 0)
    plsc.subcore_barrier()
    pltpu.sync_copy(hist_v, acc_sh.at[iota_v], add=True)
    plsc.subcore_barrier()
    pltpu.sync_copy(acc_sh, red_v)

    def bbody(i, carry):
        n0 = sid * RPT + i * L
        v = red_v[n0 // D, pl.ds(n0 % D, L)]
        for l in range(L):
            f = (i * L + l) * DEGW
            bcast_v[f // D, pl.ds(f % D, L)] = jnp.full((L,), v[l],
                                                        jnp.float32)
        return carry

    lax.fori_loop(0, RPT // L, bbody, 0)
    pltpu.sync_copy(bcast_v, stage_sh.at[pl.ds(sid * HR, HR)])
    plsc.subcore_barrier()

    @pl.when(sid == 0)
    def _():
        pltpu.sync_copy(stage_sh, out_hbm.at[cid])


CHS = 80
NCHS = EPW // CHS


NBODY = NCHS - 1


@functools.partial(
    pl.kernel,
    mesh=_mesh,
    out_type=jax.ShapeDtypeStruct((NC, NP, D), jnp.float32),
    scratch_types=[
        pltpu.VMEM((4, CHS), jnp.int32),
        pltpu.VMEM((NCHS, CHS), jnp.int32),
        pltpu.VMEM((2, CHS, D), jnp.float32),
        pltpu.VMEM_SHARED((NP, D), jnp.float32),
        pltpu.SemaphoreType.DMA,
        pltpu.SemaphoreType.DMA,
    ],
)
def _scatter_call(tab_hbm, src1d_hbm, dst3d_hbm, zeros_hbm, out_hbm,
                  si_v, didx_v, rows_v, acc_sh, semg0, semg1):
    semg = (semg0, semg1)
    cid = lax.axis_index("c")
    sid = lax.axis_index("s")
    wid = cid * NS + sid
    pltpu.sync_copy(zeros_hbm, acc_sh.at[pl.ds(sid * RPT, RPT)])
    pltpu.sync_copy(dst3d_hbm.at[wid], didx_v)
    base0 = wid * EPW
    pltpu.sync_copy(src1d_hbm.at[pl.ds(base0, CHS)], si_v.at[0])
    pltpu.sync_copy(src1d_hbm.at[pl.ds(base0 + CHS, CHS)], si_v.at[1])
    plsc.subcore_barrier()

    pltpu.async_copy(tab_hbm.at[si_v.at[0]], rows_v.at[0], semg[0])
    pltpu.async_copy(tab_hbm.at[si_v.at[1]], rows_v.at[1], semg[1])

    def body(k, carry):
        for j in range(4):
            c = k * 4 + j
            b = j % 2
            s = (j + 2) % 4

            @pl.when(c + 2 < NCHS)
            def _():
                pltpu.sync_copy(
                    src1d_hbm.at[pl.ds(base0 + (c + 2) * CHS, CHS)],
                    si_v.at[s])

            pltpu.make_async_copy(tab_hbm.at[si_v.at[0]], rows_v.at[b],
                                  semg[b]).wait()
            pltpu.sync_copy(rows_v.at[b], acc_sh.at[didx_v.at[c]], add=True)

            @pl.when(c + 2 < NCHS)
            def _():
                pltpu.async_copy(tab_hbm.at[si_v.at[s]], rows_v.at[b],
                                 semg[b])
        return carry

    lax.fori_loop(0, NBODY // 4, body, 0)
    pltpu.make_async_copy(tab_hbm.at[si_v.at[0]], rows_v.at[0],
                          semg[0]).wait()
    pltpu.sync_copy(rows_v.at[0], acc_sh.at[didx_v.at[NCHS - 1]], add=True)
    plsc.subcore_barrier()
    pltpu.sync_copy(acc_sh.at[pl.ds(sid * RPT, RPT)],
                    out_hbm.at[cid, pl.ds(sid * RPT, RPT)])


BLK = 400


def _dis_from(degp_ref):
    deg = degp_ref[0, :, 0:1] + degp_ref[1, :, 0:1]
    return jnp.where(deg > 0, lax.rsqrt(deg), 0.0)


def _mm_body(x_ref, w_ref, o_ref):
    o_ref[...] = jnp.dot(x_ref[...], w_ref[...],
                         preferred_element_type=jnp.float32,
                         precision=lax.Precision.HIGHEST)


def _scale_body(xw_ref, degp_ref, o_ref):
    o_ref[...] = xw_ref[...] * _dis_from(degp_ref)


def _combine_mm_body(p_ref, degp_ref, b_ref, w_ref, o_ref):
    dis = _dis_from(degp_ref)
    h = jnp.maximum((p_ref[0] + p_ref[1]) * dis + b_ref[...], 0.0)
    o_ref[...] = jnp.dot(h, w_ref[...],
                         preferred_element_type=jnp.float32,
                         precision=lax.Precision.HIGHEST) * dis


def _final_body(p_ref, degp_ref, b_ref, o_ref):
    dis = _dis_from(degp_ref)
    o_ref[...] = (p_ref[0] + p_ref[1]) * dis + b_ref[...]


_mm = pl.pallas_call(
    _mm_body,
    grid=(N // BLK,),
    in_specs=[
        pl.BlockSpec((BLK, D), lambda i: (i, 0)),
        pl.BlockSpec((D, D), lambda i: (0, 0)),
    ],
    out_specs=pl.BlockSpec((BLK, D), lambda i: (i, 0)),
    out_shape=jax.ShapeDtypeStruct((N, D), jnp.float32),
)

_scale = pl.pallas_call(
    _scale_body,
    grid=(N // BLK,),
    in_specs=[
        pl.BlockSpec((BLK, D), lambda i: (i, 0)),
        pl.BlockSpec((NC, BLK, DEGW), lambda i: (0, i, 0)),
    ],
    out_specs=pl.BlockSpec((BLK, D), lambda i: (i, 0)),
    out_shape=jax.ShapeDtypeStruct((N, D), jnp.float32),
)

_combine_mm = pl.pallas_call(
    _combine_mm_body,
    grid=(N // BLK,),
    in_specs=[
        pl.BlockSpec((NC, BLK, D), lambda i: (0, i, 0)),
        pl.BlockSpec((NC, BLK, DEGW), lambda i: (0, i, 0)),
        pl.BlockSpec((1, D), lambda i: (0, 0)),
        pl.BlockSpec((D, D), lambda i: (0, 0)),
    ],
    out_specs=pl.BlockSpec((BLK, D), lambda i: (i, 0)),
    out_shape=jax.ShapeDtypeStruct((N, D), jnp.float32),
)

_final = pl.pallas_call(
    _final_body,
    grid=(N // BLK,),
    in_specs=[
        pl.BlockSpec((NC, BLK, D), lambda i: (0, i, 0)),
        pl.BlockSpec((NC, BLK, DEGW), lambda i: (0, i, 0)),
        pl.BlockSpec((1, D), lambda i: (0, 0)),
    ],
    out_specs=pl.BlockSpec((BLK, D), lambda i: (i, 0)),
    out_shape=jax.ShapeDtypeStruct((N, D), jnp.float32),
)


def kernel(x, edge_index, W1, b1, W2, b2):
    src = edge_index[0].astype(jnp.int32).reshape(NW, EPW)
    dst = edge_index[1].astype(jnp.int32).reshape(NW, EPW)
    padn = EPW2 - EPW
    srcp = jnp.concatenate(
        [src, jnp.zeros((NW, padn), jnp.int32)], axis=1)
    pad_rows = N + jnp.arange(padn, dtype=jnp.int32)
    dstp = jnp.concatenate(
        [dst, jnp.broadcast_to(pad_rows, (NW, padn))], axis=1)
    src1d = edge_index[0].astype(jnp.int32)
    dst3d = edge_index[1].astype(jnp.int32).reshape(NW, NCHS, CHS)
    dst2d = dstp.reshape(NW, EPW2 // D, D)
    zeros_d = jnp.zeros((RPT, D), jnp.float32)
    b1r = b1.reshape(1, D)
    b2r = b2.reshape(1, D)

    xw1 = _mm(x, W1)
    degp = _deg_call(dst2d, zeros_d).reshape(NC, NP, DEGW)
    xws1 = _scale(xw1, degp)
    p1 = _scatter_call(xws1, src1d, dst3d, zeros_d)
    xws2 = _combine_mm(p1, degp, b1r, W2)
    p2 = _scatter_call(xws2, src1d, dst3d, zeros_d)
    return _final(p2, degp, b2r)


# speedup vs baseline: 1.0271x; 1.0260x over previous
import functools

import jax
import jax.numpy as jnp
from jax import lax
from jax.experimental import pallas as pl
from jax.experimental.pallas import tpu as pltpu
from jax.experimental.pallas import tpu_sc as plsc

N = 10000
NP = 10240
D = 128
E = 320000
NC = 2
NS = 16
NW = NC * NS
EPW = E // NW
EPW2 = 10240
CH = 128
NCHUNK = EPW2 // CH
G = 10
NGRP = NCHUNK // G
RPT = NP // NS
DEGW = 16
L = 16
HR = NP // D

_mesh = plsc.VectorSubcoreMesh(core_axis_name="c", subcore_axis_name="s")


@functools.partial(
    pl.kernel,
    mesh=_mesh,
    out_type=jax.ShapeDtypeStruct((NC, NP // 8, D), jnp.float32),
    scratch_types=[
        pltpu.VMEM((EPW2 // D, D), jnp.int32),
        pltpu.VMEM((HR, D), jnp.float32),
        pltpu.VMEM((HR,), jnp.int32),
        pltpu.VMEM((HR, D), jnp.float32),
        pltpu.VMEM((HR, D), jnp.float32),
        pltpu.VMEM_SHARED((HR, D), jnp.float32),
        pltpu.VMEM_SHARED((NP // 8, D), jnp.float32),
    ],
    compiler_params=pltpu.CompilerParams(needs_layout_passes=False),
)
def _deg_call(dst2d_hbm, zeros_hbm, out_hbm, idx_v, hist_v, iota_v, red_v,
              bcast_v, acc_sh, stage_sh):
    cid = lax.axis_index("c")
    sid = lax.axis_index("s")
    wid = cid * NS + sid
    pltpu.sync_copy(dst2d_hbm.at[wid], idx_v)

    @pl.when(sid == 0)
    def _():
        pltpu.sync_copy(zeros_hbm.at[pl.ds(0, HR)], acc_sh)

    for k in range(HR // L):
        iota_v[pl.ds(k * L, L)] = lax.iota(jnp.int32, L) + (k * L)

    def z16(i, carry):
        for k in range(D // L):
            hist_v[i, pl.ds(k * L, L)] = jnp.zeros((L,), jnp.float32)
        return carry

    lax.fori_loop(0, HR, z16, 0)

    ones16 = jnp.full((L,), 1.0, jnp.float32)

    def body(i, carry):
        for k in range(D // L):
            idx16 = idx_v[i, pl.ds(k * L, L)]
            hi = lax.shift_right_logical(idx16, 7)
            lo = lax.bitwise_and(idx16, 127)
            plsc.addupdate_scatter(hist_v, [hi, lo], ones16)
        return carry

    lax.fori_loop(0, EPW2 // D, body, 0)
    plsc.subcore_barrier()
    pltpu.sync_copy(hist_v, acc_sh.at[iota_v], add=True)
    plsc.subcore_barrier()
    pltpu.sync_copy(acc_sh, red_v)

    def bbody(i, carry):
        n0 = sid * RPT + i * L
        v = red_v[n0 // D, pl.ds(n0 % D, L)]
        for l in range(L):
            f = (i * L + l) * DEGW
            bcast_v[f // D, pl.ds(f % D, L)] = jnp.full((L,), v[l],
                                                        jnp.float32)
        return carry

    lax.fori_loop(0, RPT // L, bbody, 0)
    pltpu.sync_copy(bcast_v, stage_sh.at[pl.ds(sid * HR, HR)])
    plsc.subcore_barrier()

    @pl.when(sid == 0)
    def _():
        pltpu.sync_copy(stage_sh, out_hbm.at[cid])


CHS = 80
NCHS = EPW // CHS


NBODY = NCHS - 1


@functools.partial(
    pl.kernel,
    mesh=_mesh,
    out_type=jax.ShapeDtypeStruct((NC, NP, D), jnp.float32),
    scratch_types=[
        pltpu.VMEM((4, CHS), jnp.int32),
        pltpu.VMEM((NCHS, CHS), jnp.int32),
        pltpu.VMEM((2, CHS, D), jnp.float32),
        pltpu.VMEM_SHARED((NP, D), jnp.float32),
        pltpu.SemaphoreType.DMA,
        pltpu.SemaphoreType.DMA,
    ],
)
def _scatter_call(tab_hbm, src1d_hbm, dst3d_hbm, zeros_hbm, out_hbm,
                  si_v, didx_v, rows_v, acc_sh, semg0, semg1):
    semg = (semg0, semg1)
    cid = lax.axis_index("c")
    sid = lax.axis_index("s")
    wid = cid * NS + sid
    pltpu.sync_copy(zeros_hbm, acc_sh.at[pl.ds(sid * RPT, RPT)])
    pltpu.sync_copy(dst3d_hbm.at[wid], didx_v)
    base0 = wid * EPW
    pltpu.sync_copy(src1d_hbm.at[pl.ds(base0, CHS)], si_v.at[0])
    pltpu.sync_copy(src1d_hbm.at[pl.ds(base0 + CHS, CHS)], si_v.at[1])
    plsc.subcore_barrier()

    pltpu.async_copy(tab_hbm.at[si_v.at[0]], rows_v.at[0], semg[0])
    pltpu.async_copy(tab_hbm.at[si_v.at[1]], rows_v.at[1], semg[1])

    def body(k, carry):
        for j in range(4):
            c = k * 4 + j
            b = j % 2
            s = (j + 2) % 4

            @pl.when(c + 2 < NCHS)
            def _():
                pltpu.sync_copy(
                    src1d_hbm.at[pl.ds(base0 + (c + 2) * CHS, CHS)],
                    si_v.at[s])

            pltpu.make_async_copy(tab_hbm.at[si_v.at[0]], rows_v.at[b],
                                  semg[b]).wait()
            pltpu.sync_copy(rows_v.at[b], acc_sh.at[didx_v.at[c]], add=True)

            @pl.when(c + 2 < NCHS)
            def _():
                pltpu.async_copy(tab_hbm.at[si_v.at[s]], rows_v.at[b],
                                 semg[b])
        return carry

    lax.fori_loop(0, NBODY // 4, body, 0)
    pltpu.make_async_copy(tab_hbm.at[si_v.at[0]], rows_v.at[0],
                          semg[0]).wait()
    pltpu.sync_copy(rows_v.at[0], acc_sh.at[didx_v.at[NCHS - 1]], add=True)
    plsc.subcore_barrier()
    pltpu.sync_copy(acc_sh.at[pl.ds(sid * RPT, RPT)],
                    out_hbm.at[cid, pl.ds(sid * RPT, RPT)])


BLK = 400


def _dis_from(degp_ref):
    deg = degp_ref[0, :, 0:1] + degp_ref[1, :, 0:1]
    return jnp.where(deg > 0, lax.rsqrt(deg), 0.0)


def _mm_scale_body(x_ref, w_ref, degp_ref, o_ref):
    dis = _dis_from(degp_ref)
    xw = jnp.dot(x_ref[...], w_ref[...],
                 preferred_element_type=jnp.float32,
                 precision=lax.Precision.HIGHEST)
    o_ref[...] = xw * dis


def _combine_mm_body(p_ref, degp_ref, b_ref, w_ref, o_ref):
    dis = _dis_from(degp_ref)
    h = jnp.maximum((p_ref[0] + p_ref[1]) * dis + b_ref[...], 0.0)
    o_ref[...] = jnp.dot(h, w_ref[...],
                         preferred_element_type=jnp.float32,
                         precision=lax.Precision.HIGHEST) * dis


def _final_body(p_ref, degp_ref, b_ref, o_ref):
    dis = _dis_from(degp_ref)
    o_ref[...] = (p_ref[0] + p_ref[1]) * dis + b_ref[...]


_mm_scale = pl.pallas_call(
    _mm_scale_body,
    grid=(N // BLK,),
    in_specs=[
        pl.BlockSpec((BLK, D), lambda i: (i, 0)),
        pl.BlockSpec((D, D), lambda i: (0, 0)),
        pl.BlockSpec((NC, BLK, DEGW), lambda i: (0, i, 0)),
    ],
    out_specs=pl.BlockSpec((BLK, D), lambda i: (i, 0)),
    out_shape=jax.ShapeDtypeStruct((N, D), jnp.float32),
)

_combine_mm = pl.pallas_call(
    _combine_mm_body,
    grid=(N // BLK,),
    in_specs=[
        pl.BlockSpec((NC, BLK, D), lambda i: (0, i, 0)),
        pl.BlockSpec((NC, BLK, DEGW), lambda i: (0, i, 0)),
        pl.BlockSpec((1, D), lambda i: (0, 0)),
        pl.BlockSpec((D, D), lambda i: (0, 0)),
    ],
    out_specs=pl.BlockSpec((BLK, D), lambda i: (i, 0)),
    out_shape=jax.ShapeDtypeStruct((N, D), jnp.float32),
)

_final = pl.pallas_call(
    _final_body,
    grid=(N // BLK,),
    in_specs=[
        pl.BlockSpec((NC, BLK, D), lambda i: (0, i, 0)),
        pl.BlockSpec((NC, BLK, DEGW), lambda i: (0, i, 0)),
        pl.BlockSpec((1, D), lambda i: (0, 0)),
    ],
    out_specs=pl.BlockSpec((BLK, D), lambda i: (i, 0)),
    out_shape=jax.ShapeDtypeStruct((N, D), jnp.float32),
)


def kernel(x, edge_index, W1, b1, W2, b2):
    src = edge_index[0].astype(jnp.int32).reshape(NW, EPW)
    dst = edge_index[1].astype(jnp.int32).reshape(NW, EPW)
    padn = EPW2 - EPW
    srcp = jnp.concatenate(
        [src, jnp.zeros((NW, padn), jnp.int32)], axis=1)
    pad_rows = N + jnp.arange(padn, dtype=jnp.int32)
    dstp = jnp.concatenate(
        [dst, jnp.broadcast_to(pad_rows, (NW, padn))], axis=1)
    src1d = edge_index[0].astype(jnp.int32)
    dst3d = edge_index[1].astype(jnp.int32).reshape(NW, NCHS, CHS)
    dst2d = dstp.reshape(NW, EPW2 // D, D)
    zeros_d = jnp.zeros((RPT, D), jnp.float32)
    b1r = b1.reshape(1, D)
    b2r = b2.reshape(1, D)

    degp = _deg_call(dst2d, zeros_d).reshape(NC, NP, DEGW)
    xws1 = _mm_scale(x, W1, degp)
    p1 = _scatter_call(xws1, src1d, dst3d, zeros_d)
    xws2 = _combine_mm(p1, degp, b1r, W2)
    p2 = _scatter_call(xws2, src1d, dst3d, zeros_d)
    return _final(p2, degp, b2r)
